# baseline (device time: 132863 ns/iter reference)
import functools

import jax
import jax.numpy as jnp
from jax import lax
from jax.experimental import pallas as pl
from jax.experimental.pallas import tpu as pltpu

N_DEV = 8
STAGE_XOR = (1, 3, 4)
SCALE = 0.08838834764831843


def kernel(x, Wq, Wo, K_ext, V_ext):
    B, Sq, D = x.shape
    _, Ckv, H, Dh = K_ext.shape
    BH = B * H
    N_STAGE = len(STAGE_XOR)

    def body(x_ref, wq_ref, wo_ref, k_ref, v_ref, out_ref,
             o_state, o_send, ml_state, ml_send, comm_o, comm_ml,
             q_scratch, attn_ref, so, ro, sml, rml):
        my = lax.axis_index("i")

        barrier = pltpu.get_barrier_semaphore()
        for xr in STAGE_XOR:
            pl.semaphore_signal(
                barrier, inc=1,
                device_id=(my ^ xr,), device_id_type=pl.DeviceIdType.MESH,
            )
        pl.semaphore_wait(barrier, N_STAGE)

        def rdma_pair(s_i, c, partner):
            r_o = pltpu.make_async_remote_copy(
                src_ref=o_send.at[pl.ds(c * H, H)],
                dst_ref=comm_o.at[s_i, pl.ds(c * H, H)],
                send_sem=so.at[s_i, c],
                recv_sem=ro.at[s_i, c],
                device_id=(partner,),
                device_id_type=pl.DeviceIdType.MESH,
            )
            r_ml = pltpu.make_async_remote_copy(
                src_ref=ml_send.at[c],
                dst_ref=comm_ml.at[s_i, c],
                send_sem=sml.at[s_i, c],
                recv_sem=rml.at[s_i, c],
                device_id=(partner,),
                device_id_type=pl.DeviceIdType.MESH,
            )
            return r_o, r_ml

        x_all = jnp.reshape(x_ref[:], (B * Sq, D)).astype(jnp.bfloat16)
        wq = wq_ref[:].astype(jnp.bfloat16)
        q_all = lax.dot_general(
            x_all, wq, (((1,), (0,)), ((), ())),
            preferred_element_type=jnp.float32,
        )
        q_scratch[:] = (q_all * SCALE).astype(jnp.bfloat16)

        def local_b(b, carry):
            q = q_scratch[pl.ds(b * Sq, Sq), :]
            for h in range(H):
                qh = q[:, h * Dh:(h + 1) * Dh]
                kh = k_ref[b, :, h, :]
                vh = v_ref[b, :, h, :]
                s = lax.dot_general(
                    qh, kh, (((1,), (1,)), ((), ())),
                    preferred_element_type=jnp.float32,
                )
                m = jnp.max(s, axis=1, keepdims=True)
                p = jnp.exp(s - m)
                l = jnp.sum(p, axis=1, keepdims=True)
                o = lax.dot_general(
                    p.astype(jnp.bfloat16), vh, (((1,), (0,)), ((), ())),
                    preferred_element_type=jnp.float32,
                )
                o_state[b * H + h] = o
                o_send[b * H + h] = o.astype(jnp.bfloat16)
                ml_state[b, :, h:h + 1] = m
                ml_state[b, :, H + h:H + h + 1] = l
                ml_send[b, :, h:h + 1] = m.astype(jnp.bfloat16)
                ml_send[b, :, H + h:H + h + 1] = l.astype(jnp.bfloat16)
            return carry

        lax.fori_loop(0, B // 2, local_b, 0)
        for c in range(B // 2):
            r_o, r_ml = rdma_pair(0, c, my ^ STAGE_XOR[0])
            r_o.start()
            r_ml.start()
        lax.fori_loop(B // 2, B, local_b, 0)
        for c in range(B // 2, B):
            r_o, r_ml = rdma_pair(0, c, my ^ STAGE_XOR[0])
            r_o.start()
            r_ml.start()

        for s_i, xr in enumerate(STAGE_XOR):
            last = s_i == N_STAGE - 1
            for c in range(B):
                r_o, r_ml = rdma_pair(s_i, c, my ^ xr)
                r_o.wait()
                r_ml.wait()

                def merge_b(b, carry, s_i=s_i, last=last):
                    m_mine = ml_state[b, :, 0:H]
                    l_mine = ml_state[b, :, H:2 * H]
                    m_oth = comm_ml[s_i, b, :, 0:H].astype(jnp.float32)
                    l_oth = comm_ml[s_i, b, :, H:2 * H].astype(jnp.float32)
                    m_new = jnp.maximum(m_mine, m_oth)
                    a_mine = jnp.exp(m_mine - m_new)
                    a_oth = jnp.exp(m_oth - m_new)
                    l_new = l_mine * a_mine + l_oth * a_oth
                    ml_state[b, :, 0:H] = m_new
                    ml_state[b, :, H:2 * H] = l_new
                    if not last:
                        ml_send[b, :, 0:H] = m_new.astype(jnp.bfloat16)
                        ml_send[b, :, H:2 * H] = l_new.astype(jnp.bfloat16)
                    for h in range(H):
                        idx = b * H + h
                        merged = (
                            o_state[idx] * a_mine[:, h:h + 1]
                            + comm_o[s_i, idx].astype(jnp.float32)
                            * a_oth[:, h:h + 1]
                        )
                        o_state[idx] = merged
                        if not last:
                            o_send[idx] = merged.astype(jnp.bfloat16)
                    return carry

                lax.fori_loop(c, c + 1, merge_b, 0)
                if not last:
                    n_o, n_ml = rdma_pair(s_i + 1, c, my ^ STAGE_XOR[s_i + 1])
                    n_o.start()
                    n_ml.start()

        def final_b(b, carry):
            for h in range(H):
                inv_l = 1.0 / ml_state[b, :, H + h:H + h + 1]
                attn_ref[pl.ds(b * Sq, Sq), h * Dh:(h + 1) * Dh] = (
                    o_state[b * H + h] * inv_l
                ).astype(jnp.bfloat16)
            return carry

        lax.fori_loop(0, B, final_b, 0)
        wo = wo_ref[:].astype(jnp.bfloat16)
        out = lax.dot_general(
            attn_ref[:], wo, (((1,), (0,)), ((), ())),
            preferred_element_type=jnp.float32,
        )
        out_ref[:] = jnp.reshape(out, (B, Sq, D))

        @functools.partial(
            pl.run_scoped, second_barrier=pltpu.SemaphoreType.REGULAR
        )
        def _(second_barrier):
            for xr in STAGE_XOR:
                pl.semaphore_signal(
                    second_barrier, inc=1,
                    device_id=(my ^ xr,), device_id_type=pl.DeviceIdType.MESH,
                )
            pl.semaphore_wait(second_barrier, N_STAGE)

    return pl.pallas_call(
        body,
        out_shape=jax.ShapeDtypeStruct((B, Sq, D), jnp.float32),
        in_specs=[pl.BlockSpec(memory_space=pltpu.VMEM)] * 5,
        out_specs=pl.BlockSpec(memory_space=pltpu.VMEM),
        scratch_shapes=[
            pltpu.VMEM((BH, Sq, Dh), jnp.float32),
            pltpu.VMEM((BH, Sq, Dh), jnp.bfloat16),
            pltpu.VMEM((B, Sq, 2 * H), jnp.float32),
            pltpu.VMEM((B, Sq, 2 * H), jnp.bfloat16),
            pltpu.VMEM((N_STAGE, BH, Sq, Dh), jnp.bfloat16),
            pltpu.VMEM((N_STAGE, B, Sq, 2 * H), jnp.bfloat16),
            pltpu.VMEM((B * Sq, D), jnp.bfloat16),
            pltpu.VMEM((B * Sq, D), jnp.bfloat16),
            pltpu.SemaphoreType.DMA((N_STAGE, B)),
            pltpu.SemaphoreType.DMA((N_STAGE, B)),
            pltpu.SemaphoreType.DMA((N_STAGE, B)),
            pltpu.SemaphoreType.DMA((N_STAGE, B)),
        ],
        compiler_params=pltpu.CompilerParams(
            collective_id=0, vmem_limit_bytes=63 * 1024 * 1024
        ),
    )(x, Wq, Wo, K_ext.astype(jnp.bfloat16), V_ext.astype(jnp.bfloat16))


# device time: 117601 ns/iter; 1.1298x vs baseline; 1.1298x over previous
import functools

import jax
import jax.numpy as jnp
from jax import lax
from jax.experimental import pallas as pl
from jax.experimental.pallas import tpu as pltpu

N_DEV = 8
STAGE_XOR = (1, 3, 4)
SCALE = 0.08838834764831843


def kernel(x, Wq, Wo, K_ext, V_ext):
    B, Sq, D = x.shape
    _, Ckv, H, Dh = K_ext.shape
    BH = B * H
    N_STAGE = len(STAGE_XOR)

    def body(x_ref, wq_ref, wo_ref, k_ref, v_ref, out_ref,
             o_state, o_send, ml_state, ml_send, comm_o, comm_ml,
             q_scratch, attn_ref, so, ro, sml, rml):
        my = lax.axis_index("i")

        barrier = pltpu.get_barrier_semaphore()
        for xr in STAGE_XOR:
            pl.semaphore_signal(
                barrier, inc=1,
                device_id=(my ^ xr,), device_id_type=pl.DeviceIdType.MESH,
            )
        pl.semaphore_wait(barrier, N_STAGE)

        def rdma_pair(s_i, c, partner):
            r_o = pltpu.make_async_remote_copy(
                src_ref=o_send.at[pl.ds(c * H, H)],
                dst_ref=comm_o.at[s_i, pl.ds(c * H, H)],
                send_sem=so.at[s_i, c],
                recv_sem=ro.at[s_i, c],
                device_id=(partner,),
                device_id_type=pl.DeviceIdType.MESH,
            )
            r_ml = pltpu.make_async_remote_copy(
                src_ref=ml_send.at[c],
                dst_ref=comm_ml.at[s_i, c],
                send_sem=sml.at[s_i, c],
                recv_sem=rml.at[s_i, c],
                device_id=(partner,),
                device_id_type=pl.DeviceIdType.MESH,
            )
            return r_o, r_ml

        x_all = jnp.reshape(x_ref[:], (B * Sq, D)).astype(jnp.bfloat16)
        wq = wq_ref[:].astype(jnp.bfloat16)
        q_all = lax.dot_general(
            x_all, wq, (((1,), (0,)), ((), ())),
            preferred_element_type=jnp.float32,
        )
        q_scratch[:] = (q_all * SCALE).astype(jnp.bfloat16)

        def local_b(b, carry):
            q = q_scratch[pl.ds(b * Sq, Sq), :]
            for h in range(H):
                qh = q[:, h * Dh:(h + 1) * Dh]
                kh = k_ref[b, :, h * Dh:(h + 1) * Dh]
                vh = v_ref[b, :, h * Dh:(h + 1) * Dh]
                s = lax.dot_general(
                    qh, kh, (((1,), (1,)), ((), ())),
                    preferred_element_type=jnp.float32,
                )
                m = jnp.max(s, axis=1, keepdims=True)
                p = jnp.exp(s - m)
                l = jnp.sum(p, axis=1, keepdims=True)
                o = lax.dot_general(
                    p.astype(jnp.bfloat16), vh, (((1,), (0,)), ((), ())),
                    preferred_element_type=jnp.float32,
                )
                o_state[b * H + h] = o
                o_send[b * H + h] = o.astype(jnp.bfloat16)
                ml_state[b, :, h:h + 1] = m
                ml_state[b, :, H + h:H + h + 1] = l
                ml_send[b, :, h:h + 1] = m.astype(jnp.bfloat16)
                ml_send[b, :, H + h:H + h + 1] = l.astype(jnp.bfloat16)
            return carry

        lax.fori_loop(0, B // 2, local_b, 0)
        for c in range(B // 2):
            r_o, r_ml = rdma_pair(0, c, my ^ STAGE_XOR[0])
            r_o.start()
            r_ml.start()
        lax.fori_loop(B // 2, B, local_b, 0)
        for c in range(B // 2, B):
            r_o, r_ml = rdma_pair(0, c, my ^ STAGE_XOR[0])
            r_o.start()
            r_ml.start()

        for s_i, xr in enumerate(STAGE_XOR):
            last = s_i == N_STAGE - 1
            for c in range(B):
                r_o, r_ml = rdma_pair(s_i, c, my ^ xr)
                r_o.wait()
                r_ml.wait()

                def merge_b(b, carry, s_i=s_i, last=last):
                    m_mine = ml_state[b, :, 0:H]
                    l_mine = ml_state[b, :, H:2 * H]
                    m_oth = comm_ml[s_i, b, :, 0:H].astype(jnp.float32)
                    l_oth = comm_ml[s_i, b, :, H:2 * H].astype(jnp.float32)
                    m_new = jnp.maximum(m_mine, m_oth)
                    a_mine = jnp.exp(m_mine - m_new)
                    a_oth = jnp.exp(m_oth - m_new)
                    l_new = l_mine * a_mine + l_oth * a_oth
                    ml_state[b, :, 0:H] = m_new
                    ml_state[b, :, H:2 * H] = l_new
                    if not last:
                        ml_send[b, :, 0:H] = m_new.astype(jnp.bfloat16)
                        ml_send[b, :, H:2 * H] = l_new.astype(jnp.bfloat16)
                    for h in range(H):
                        idx = b * H + h
                        merged = (
                            o_state[idx] * a_mine[:, h:h + 1]
                            + comm_o[s_i, idx].astype(jnp.float32)
                            * a_oth[:, h:h + 1]
                        )
                        o_state[idx] = merged
                        if not last:
                            o_send[idx] = merged.astype(jnp.bfloat16)
                    return carry

                lax.fori_loop(c, c + 1, merge_b, 0)
                if not last:
                    n_o, n_ml = rdma_pair(s_i + 1, c, my ^ STAGE_XOR[s_i + 1])
                    n_o.start()
                    n_ml.start()

        def final_b(b, carry):
            for h in range(H):
                inv_l = 1.0 / ml_state[b, :, H + h:H + h + 1]
                attn_ref[pl.ds(b * Sq, Sq), h * Dh:(h + 1) * Dh] = (
                    o_state[b * H + h] * inv_l
                ).astype(jnp.bfloat16)
            return carry

        lax.fori_loop(0, B, final_b, 0)
        wo = wo_ref[:].astype(jnp.bfloat16)
        out = lax.dot_general(
            attn_ref[:], wo, (((1,), (0,)), ((), ())),
            preferred_element_type=jnp.float32,
        )
        out_ref[:] = jnp.reshape(out, (B, Sq, D))

        @functools.partial(
            pl.run_scoped, second_barrier=pltpu.SemaphoreType.REGULAR
        )
        def _(second_barrier):
            for xr in STAGE_XOR:
                pl.semaphore_signal(
                    second_barrier, inc=1,
                    device_id=(my ^ xr,), device_id_type=pl.DeviceIdType.MESH,
                )
            pl.semaphore_wait(second_barrier, N_STAGE)

    return pl.pallas_call(
        body,
        out_shape=jax.ShapeDtypeStruct((B, Sq, D), jnp.float32),
        in_specs=[pl.BlockSpec(memory_space=pltpu.VMEM)] * 5,
        out_specs=pl.BlockSpec(memory_space=pltpu.VMEM),
        scratch_shapes=[
            pltpu.VMEM((BH, Sq, Dh), jnp.float32),
            pltpu.VMEM((BH, Sq, Dh), jnp.bfloat16),
            pltpu.VMEM((B, Sq, 2 * H), jnp.float32),
            pltpu.VMEM((B, Sq, 2 * H), jnp.bfloat16),
            pltpu.VMEM((N_STAGE, BH, Sq, Dh), jnp.bfloat16),
            pltpu.VMEM((N_STAGE, B, Sq, 2 * H), jnp.bfloat16),
            pltpu.VMEM((B * Sq, D), jnp.bfloat16),
            pltpu.VMEM((B * Sq, D), jnp.bfloat16),
            pltpu.SemaphoreType.DMA((N_STAGE, B)),
            pltpu.SemaphoreType.DMA((N_STAGE, B)),
            pltpu.SemaphoreType.DMA((N_STAGE, B)),
            pltpu.SemaphoreType.DMA((N_STAGE, B)),
        ],
        compiler_params=pltpu.CompilerParams(
            collective_id=0, vmem_limit_bytes=63 * 1024 * 1024
        ),
    )(
        x, Wq, Wo,
        jnp.reshape(K_ext.astype(jnp.bfloat16), (B, Ckv, H * Dh)),
        jnp.reshape(V_ext.astype(jnp.bfloat16), (B, Ckv, H * Dh)),
    )


# device time: 113052 ns/iter; 1.1752x vs baseline; 1.0402x over previous
import functools

import jax
import jax.numpy as jnp
from jax import lax
from jax.experimental import pallas as pl
from jax.experimental.pallas import tpu as pltpu

N_DEV = 8
STAGE_XOR = (1, 3, 4)
SCALE = 0.08838834764831843


def kernel(x, Wq, Wo, K_ext, V_ext):
    B, Sq, D = x.shape
    _, Ckv, H, Dh = K_ext.shape
    BH = B * H
    N_STAGE = len(STAGE_XOR)

    def body(x_ref, wq_ref, wo_ref, k_ref, v_ref, out_ref,
             o_state, o_send, ml_state, ml_send, comm_o, comm_ml,
             q_scratch, attn_ref, so, ro, sml, rml):
        my = lax.axis_index("i")

        barrier = pltpu.get_barrier_semaphore()
        for xr in STAGE_XOR:
            pl.semaphore_signal(
                barrier, inc=1,
                device_id=(my ^ xr,), device_id_type=pl.DeviceIdType.MESH,
            )
        pl.semaphore_wait(barrier, N_STAGE)

        def rdma_pair(s_i, c, partner):
            r_o = pltpu.make_async_remote_copy(
                src_ref=o_send.at[pl.ds(c * H, H)],
                dst_ref=comm_o.at[s_i, pl.ds(c * H, H)],
                send_sem=so.at[s_i, c],
                recv_sem=ro.at[s_i, c],
                device_id=(partner,),
                device_id_type=pl.DeviceIdType.MESH,
            )
            r_ml = pltpu.make_async_remote_copy(
                src_ref=ml_send.at[c],
                dst_ref=comm_ml.at[s_i, c],
                send_sem=sml.at[s_i, c],
                recv_sem=rml.at[s_i, c],
                device_id=(partner,),
                device_id_type=pl.DeviceIdType.MESH,
            )
            return r_o, r_ml

        x_all = jnp.reshape(x_ref[:], (B * Sq, D)).astype(jnp.bfloat16)
        wq = wq_ref[:].astype(jnp.bfloat16)
        q_all = lax.dot_general(
            x_all, wq, (((1,), (0,)), ((), ())),
            preferred_element_type=jnp.float32,
        )
        q_scratch[:] = (q_all * SCALE).astype(jnp.bfloat16)

        def local_b(b, carry):
            q = q_scratch[pl.ds(b * Sq, Sq), :]
            for h in range(H):
                qh = q[:, h * Dh:(h + 1) * Dh]
                kh = k_ref[b, h]
                vh = v_ref[b, h]
                s = lax.dot_general(
                    qh, kh, (((1,), (1,)), ((), ())),
                    preferred_element_type=jnp.float32,
                )
                m = jnp.max(s, axis=1, keepdims=True)
                p = jnp.exp(s - m)
                l = jnp.sum(p, axis=1, keepdims=True)
                o = lax.dot_general(
                    p.astype(jnp.bfloat16), vh, (((1,), (0,)), ((), ())),
                    preferred_element_type=jnp.float32,
                )
                o_state[b * H + h] = o
                o_send[b * H + h] = o.astype(jnp.bfloat16)
                ml_state[b, :, h:h + 1] = m
                ml_state[b, :, H + h:H + h + 1] = l
                ml_send[b, :, h:h + 1] = m.astype(jnp.bfloat16)
                ml_send[b, :, H + h:H + h + 1] = l.astype(jnp.bfloat16)
            return carry

        lax.fori_loop(0, B // 2, local_b, 0)
        for c in range(B // 2):
            r_o, r_ml = rdma_pair(0, c, my ^ STAGE_XOR[0])
            r_o.start()
            r_ml.start()
        lax.fori_loop(B // 2, B, local_b, 0)
        for c in range(B // 2, B):
            r_o, r_ml = rdma_pair(0, c, my ^ STAGE_XOR[0])
            r_o.start()
            r_ml.start()

        for s_i, xr in enumerate(STAGE_XOR):
            last = s_i == N_STAGE - 1
            for c in range(B):
                r_o, r_ml = rdma_pair(s_i, c, my ^ xr)
                r_o.wait()
                r_ml.wait()

                def merge_b(b, carry, s_i=s_i, last=last):
                    m_mine = ml_state[b, :, 0:H]
                    l_mine = ml_state[b, :, H:2 * H]
                    m_oth = comm_ml[s_i, b, :, 0:H].astype(jnp.float32)
                    l_oth = comm_ml[s_i, b, :, H:2 * H].astype(jnp.float32)
                    m_new = jnp.maximum(m_mine, m_oth)
                    a_mine = jnp.exp(m_mine - m_new)
                    a_oth = jnp.exp(m_oth - m_new)
                    l_new = l_mine * a_mine + l_oth * a_oth
                    ml_state[b, :, 0:H] = m_new
                    ml_state[b, :, H:2 * H] = l_new
                    if not last:
                        ml_send[b, :, 0:H] = m_new.astype(jnp.bfloat16)
                        ml_send[b, :, H:2 * H] = l_new.astype(jnp.bfloat16)
                    for h in range(H):
                        idx = b * H + h
                        merged = (
                            o_state[idx] * a_mine[:, h:h + 1]
                            + comm_o[s_i, idx].astype(jnp.float32)
                            * a_oth[:, h:h + 1]
                        )
                        o_state[idx] = merged
                        if not last:
                            o_send[idx] = merged.astype(jnp.bfloat16)
                    return carry

                lax.fori_loop(c, c + 1, merge_b, 0)
                if not last:
                    n_o, n_ml = rdma_pair(s_i + 1, c, my ^ STAGE_XOR[s_i + 1])
                    n_o.start()
                    n_ml.start()

        def final_b(b, carry):
            for h in range(H):
                inv_l = 1.0 / ml_state[b, :, H + h:H + h + 1]
                attn_ref[pl.ds(b * Sq, Sq), h * Dh:(h + 1) * Dh] = (
                    o_state[b * H + h] * inv_l
                ).astype(jnp.bfloat16)
            return carry

        lax.fori_loop(0, B, final_b, 0)
        wo = wo_ref[:].astype(jnp.bfloat16)
        out = lax.dot_general(
            attn_ref[:], wo, (((1,), (0,)), ((), ())),
            preferred_element_type=jnp.float32,
        )
        out_ref[:] = jnp.reshape(out, (B, Sq, D))

        @functools.partial(
            pl.run_scoped, second_barrier=pltpu.SemaphoreType.REGULAR
        )
        def _(second_barrier):
            for xr in STAGE_XOR:
                pl.semaphore_signal(
                    second_barrier, inc=1,
                    device_id=(my ^ xr,), device_id_type=pl.DeviceIdType.MESH,
                )
            pl.semaphore_wait(second_barrier, N_STAGE)

    return pl.pallas_call(
        body,
        out_shape=jax.ShapeDtypeStruct((B, Sq, D), jnp.float32),
        in_specs=[pl.BlockSpec(memory_space=pltpu.VMEM)] * 5,
        out_specs=pl.BlockSpec(memory_space=pltpu.VMEM),
        scratch_shapes=[
            pltpu.VMEM((BH, Sq, Dh), jnp.float32),
            pltpu.VMEM((BH, Sq, Dh), jnp.bfloat16),
            pltpu.VMEM((B, Sq, 2 * H), jnp.float32),
            pltpu.VMEM((B, Sq, 2 * H), jnp.bfloat16),
            pltpu.VMEM((N_STAGE, BH, Sq, Dh), jnp.bfloat16),
            pltpu.VMEM((N_STAGE, B, Sq, 2 * H), jnp.bfloat16),
            pltpu.VMEM((B * Sq, D), jnp.bfloat16),
            pltpu.VMEM((B * Sq, D), jnp.bfloat16),
            pltpu.SemaphoreType.DMA((N_STAGE, B)),
            pltpu.SemaphoreType.DMA((N_STAGE, B)),
            pltpu.SemaphoreType.DMA((N_STAGE, B)),
            pltpu.SemaphoreType.DMA((N_STAGE, B)),
        ],
        compiler_params=pltpu.CompilerParams(
            collective_id=0, vmem_limit_bytes=63 * 1024 * 1024
        ),
    )(
        x, Wq, Wo,
        K_ext.astype(jnp.bfloat16).transpose(0, 2, 1, 3),
        V_ext.astype(jnp.bfloat16).transpose(0, 2, 1, 3),
    )


# device time: 108669 ns/iter; 1.2226x vs baseline; 1.0403x over previous
import functools

import jax
import jax.numpy as jnp
from jax import lax
from jax.experimental import pallas as pl
from jax.experimental.pallas import tpu as pltpu

N_DEV = 8
STAGE_XOR = (1, 3, 4)
SCALE = 0.08838834764831843


def kernel(x, Wq, Wo, K_ext, V_ext):
    B, Sq, D = x.shape
    _, Ckv, H, Dh = K_ext.shape
    BH = B * H
    N_STAGE = len(STAGE_XOR)

    def body(x_ref, wq_ref, wo_ref, k_ref, v_ref, out_ref,
             o_send, ml_state, ml_send, comm_o, comm_ml,
             q_scratch, attn_ref, so, ro, sml, rml):
        my = lax.axis_index("i")

        barrier = pltpu.get_barrier_semaphore()
        for xr in STAGE_XOR:
            pl.semaphore_signal(
                barrier, inc=1,
                device_id=(my ^ xr,), device_id_type=pl.DeviceIdType.MESH,
            )
        pl.semaphore_wait(barrier, N_STAGE)

        def rdma_pair(s_i, c, partner):
            r_o = pltpu.make_async_remote_copy(
                src_ref=o_send.at[pl.ds(c * H, H)],
                dst_ref=comm_o.at[s_i, pl.ds(c * H, H)],
                send_sem=so.at[s_i, c],
                recv_sem=ro.at[s_i, c],
                device_id=(partner,),
                device_id_type=pl.DeviceIdType.MESH,
            )
            r_ml = pltpu.make_async_remote_copy(
                src_ref=ml_send.at[c],
                dst_ref=comm_ml.at[s_i, c],
                send_sem=sml.at[s_i, c],
                recv_sem=rml.at[s_i, c],
                device_id=(partner,),
                device_id_type=pl.DeviceIdType.MESH,
            )
            return r_o, r_ml

        x_all = jnp.reshape(x_ref[:], (B * Sq, D)).astype(jnp.bfloat16)
        wq = wq_ref[:].astype(jnp.bfloat16)
        q_all = lax.dot_general(
            x_all, wq, (((1,), (0,)), ((), ())),
            preferred_element_type=jnp.float32,
        )
        q_scratch[:] = (q_all * SCALE).astype(jnp.bfloat16)

        def local_b(b, carry):
            q = q_scratch[pl.ds(b * Sq, Sq), :]
            for h in range(H):
                qh = q[:, h * Dh:(h + 1) * Dh]
                kh = k_ref[b, h]
                vh = v_ref[b, h]
                s = lax.dot_general(
                    qh, kh, (((1,), (1,)), ((), ())),
                    preferred_element_type=jnp.float32,
                )
                m = jnp.max(s, axis=1, keepdims=True)
                p = jnp.exp(s - m)
                l = jnp.sum(p, axis=1, keepdims=True)
                o = lax.dot_general(
                    p.astype(jnp.bfloat16), vh, (((1,), (0,)), ((), ())),
                    preferred_element_type=jnp.float32,
                )
                o_send[b * H + h] = o.astype(jnp.bfloat16)
                ml_state[b, :, h:h + 1] = m
                ml_state[b, :, H + h:H + h + 1] = l
                ml_send[b, :, h:h + 1] = m.astype(jnp.bfloat16)
                ml_send[b, :, H + h:H + h + 1] = l.astype(jnp.bfloat16)
            return carry

        for c in range(B):
            lax.fori_loop(c, c + 1, local_b, 0)
            r_o, r_ml = rdma_pair(0, c, my ^ STAGE_XOR[0])
            r_o.start()
            r_ml.start()

        for s_i, xr in enumerate(STAGE_XOR):
            last = s_i == N_STAGE - 1
            for c in range(B):
                r_o, r_ml = rdma_pair(s_i, c, my ^ xr)
                r_o.wait()
                r_ml.wait()

                def merge_b(b, carry, s_i=s_i, last=last):
                    m_mine = ml_state[b, :, 0:H]
                    l_mine = ml_state[b, :, H:2 * H]
                    m_oth = comm_ml[s_i, b, :, 0:H].astype(jnp.float32)
                    l_oth = comm_ml[s_i, b, :, H:2 * H].astype(jnp.float32)
                    m_new = jnp.maximum(m_mine, m_oth)
                    a_mine = jnp.exp(m_mine - m_new)
                    a_oth = jnp.exp(m_oth - m_new)
                    l_new = l_mine * a_mine + l_oth * a_oth
                    ml_state[b, :, 0:H] = m_new
                    ml_state[b, :, H:2 * H] = l_new
                    if not last:
                        ml_send[b, :, 0:H] = m_new.astype(jnp.bfloat16)
                        ml_send[b, :, H:2 * H] = l_new.astype(jnp.bfloat16)
                    for h in range(H):
                        idx = b * H + h
                        merged = (
                            o_send[idx].astype(jnp.float32)
                            * a_mine[:, h:h + 1]
                            + comm_o[s_i, idx].astype(jnp.float32)
                            * a_oth[:, h:h + 1]
                        )
                        o_send[idx] = merged.astype(jnp.bfloat16)
                    return carry

                lax.fori_loop(c, c + 1, merge_b, 0)
                if not last:
                    n_o, n_ml = rdma_pair(s_i + 1, c, my ^ STAGE_XOR[s_i + 1])
                    n_o.start()
                    n_ml.start()

        def final_b(b, carry):
            for h in range(H):
                inv_l = 1.0 / ml_state[b, :, H + h:H + h + 1]
                attn_ref[pl.ds(b * Sq, Sq), h * Dh:(h + 1) * Dh] = (
                    o_send[b * H + h].astype(jnp.float32) * inv_l
                ).astype(jnp.bfloat16)
            return carry

        lax.fori_loop(0, B, final_b, 0)
        wo = wo_ref[:].astype(jnp.bfloat16)
        out = lax.dot_general(
            attn_ref[:], wo, (((1,), (0,)), ((), ())),
            preferred_element_type=jnp.float32,
        )
        out_ref[:] = jnp.reshape(out, (B, Sq, D))

        @functools.partial(
            pl.run_scoped, second_barrier=pltpu.SemaphoreType.REGULAR
        )
        def _(second_barrier):
            for xr in STAGE_XOR:
                pl.semaphore_signal(
                    second_barrier, inc=1,
                    device_id=(my ^ xr,), device_id_type=pl.DeviceIdType.MESH,
                )
            pl.semaphore_wait(second_barrier, N_STAGE)

    return pl.pallas_call(
        body,
        out_shape=jax.ShapeDtypeStruct((B, Sq, D), jnp.float32),
        in_specs=[pl.BlockSpec(memory_space=pltpu.VMEM)] * 5,
        out_specs=pl.BlockSpec(memory_space=pltpu.VMEM),
        scratch_shapes=[
            pltpu.VMEM((BH, Sq, Dh), jnp.bfloat16),
            pltpu.VMEM((B, Sq, 2 * H), jnp.float32),
            pltpu.VMEM((B, Sq, 2 * H), jnp.bfloat16),
            pltpu.VMEM((N_STAGE, BH, Sq, Dh), jnp.bfloat16),
            pltpu.VMEM((N_STAGE, B, Sq, 2 * H), jnp.bfloat16),
            pltpu.VMEM((B * Sq, D), jnp.bfloat16),
            pltpu.VMEM((B * Sq, D), jnp.bfloat16),
            pltpu.SemaphoreType.DMA((N_STAGE, B)),
            pltpu.SemaphoreType.DMA((N_STAGE, B)),
            pltpu.SemaphoreType.DMA((N_STAGE, B)),
            pltpu.SemaphoreType.DMA((N_STAGE, B)),
        ],
        compiler_params=pltpu.CompilerParams(
            collective_id=0, vmem_limit_bytes=63 * 1024 * 1024
        ),
    )(
        x, Wq, Wo,
        K_ext.astype(jnp.bfloat16).transpose(0, 2, 1, 3),
        V_ext.astype(jnp.bfloat16).transpose(0, 2, 1, 3),
    )
